# Initial kernel scaffold; baseline (speedup 1.0000x reference)
import jax
import jax.numpy as jnp
from jax.experimental import pallas as pl


def kernel(x, edge_index, edge_weight, W1, b1, W2, b2):
    raise NotImplementedError("write your pallas kernel here")


# speedup vs baseline: 13.0854x; 13.0854x over previous
import functools

import jax
import jax.numpy as jnp
from jax import lax
from jax.experimental import pallas as pl
from jax.experimental.pallas import tpu as pltpu
from jax.experimental.pallas import tpu_sc as plsc

N = 10000
NP = 10240
E = 320000
D = 128
DO = 16
NC = 2
NS = 16
K = 80
NB = 125
NDB = 250
ROWS_T = 640
ROWS_W = 320

_f32 = jnp.float32
_i32 = jnp.int32


def _zeros16():
    return jnp.zeros((16,), _f32)


def _splat_i(val):
    return jnp.full((16,), val, dtype=_i32)


def _frsqrt(d):
    i = lax.bitcast_convert_type(d, _i32)
    i = jnp.full((16,), 0x5F3759DF, dtype=_i32) - (i >> 1)
    y = lax.bitcast_convert_type(i, _f32)
    for _ in range(3):
        y = y * (1.5 - 0.5 * d * y * y)
    return y


def _sc1_body(xw1a, xw1b, srcc, dstc, ewc,
              acc1, norm, invd,
              sbuf, dbuf, wbuf, nbuf, dloc, rowb, zbuf, t640, ibuf,
              acc_sh, deg_sh, dis_sh, semg, sems):
    c = lax.axis_index("c")
    s = lax.axis_index("s")
    w = c * NS + s
    ns0 = s * ROWS_T

    def _zrow(i, _):
        for u in range(4):
            zbuf[i, pl.ds(u * 16, 16)] = _zeros16()
        return 0
    lax.fori_loop(0, 128, _zrow, 0)

    def _z640(i, _):
        t640[pl.ds(i * 16, 16)] = _zeros16()
        return 0
    lax.fori_loop(0, 40, _z640, 0)

    pltpu.sync_copy(t640, deg_sh.at[pl.ds(ns0, ROWS_T)])
    plsc.subcore_barrier()

    for v in range(2):
        pltpu.sync_copy(dstc.at[2 * s + v], dbuf)
        pltpu.sync_copy(ewc.at[2 * s + v], wbuf)
        for g in range(5):
            def _fire(b, _):
                pltpu.async_copy(wbuf.at[b], deg_sh.at[dbuf.at[b]], sems,
                                 add=True)
                return 0
            lax.fori_loop(g * 25, (g + 1) * 25, _fire, 0)

            def _drain(b, _):
                pltpu.make_async_copy(wbuf.at[0], deg_sh.at[dbuf.at[0]],
                                      sems).wait()
                return 0
            lax.fori_loop(0, 25, _drain, 0)
    plsc.subcore_barrier()

    pltpu.sync_copy(deg_sh.at[pl.ds(ns0, ROWS_T)], t640)

    def _dis(k, _):
        d16 = t640[pl.ds(k * 16, 16)] + 1.0
        y = _frsqrt(d16)
        dloc[pl.ds(ns0 + k * 16, 16)] = y
        t640[pl.ds(k * 16, 16)] = y * y
        return 0
    lax.fori_loop(0, 40, _dis, 0)
    pltpu.sync_copy(dloc.at[pl.ds(ns0, ROWS_T)], dis_sh.at[pl.ds(ns0, ROWS_T)])

    @pl.when(c == 0)
    def _():
        pltpu.sync_copy(t640, invd.at[pl.ds(ns0, ROWS_T)])

    plsc.subcore_barrier()
    pltpu.sync_copy(dis_sh, dloc)

    pltpu.sync_copy(srcc.at[w], sbuf)
    pltpu.sync_copy(dstc.at[w], dbuf)
    pltpu.sync_copy(ewc.at[w], wbuf)

    def _norm(b, _):
        for u in range(5):
            sv = sbuf[b, pl.ds(u * 16, 16)]
            dv = dbuf[b, pl.ds(u * 16, 16)]
            wv = wbuf[b, pl.ds(u * 16, 16)]
            nv = plsc.load_gather(dloc, [sv]) * wv * plsc.load_gather(dloc, [dv])
            nbuf[b, pl.ds(u * 16, 16)] = nv
        return 0
    lax.fori_loop(0, NB, _norm, 0)
    pltpu.sync_copy(nbuf, norm.at[w])

    node0 = c * (NP // 2) + s * ROWS_W
    for half, xwh in ((0, xw1a), (1, xw1b)):
        for k in range(5):
            pltpu.sync_copy(zbuf, acc_sh.at[pl.ds(ns0 + k * 128, 128), :])
        plsc.subcore_barrier()

        def _edge_batch(b, _):
            pltpu.async_copy(xwh.at[sbuf.at[b]], rowb, semg).wait()

            def _scale(jj, _):
                sc = plsc.load_gather(nbuf, [_splat_i(b), _splat_i(jj)])
                for u in range(4):
                    rowb[jj, pl.ds(u * 16, 16)] = rowb[jj, pl.ds(u * 16, 16)] * sc
                return 0
            lax.fori_loop(0, K, _scale, 0)
            pltpu.async_copy(rowb, acc_sh.at[dbuf.at[b]], sems, add=True).wait()
            return 0
        lax.fori_loop(0, NB, _edge_batch, 0)

        for k in range(4):
            base = node0 + k * K
            pltpu.sync_copy(xwh.at[pl.ds(base, K), :], rowb)
            for u in range(5):
                ibuf[k, pl.ds(u * 16, 16)] = lax.iota(_i32, 16) + (base + u * 16)

            def _selfscale(jj, _):
                sp = plsc.load_gather(dloc, [_splat_i(base + jj)])
                sp = sp * sp
                for u in range(4):
                    rowb[jj, pl.ds(u * 16, 16)] = rowb[jj, pl.ds(u * 16, 16)] * sp
                return 0
            lax.fori_loop(0, K, _selfscale, 0)
            pltpu.async_copy(rowb, acc_sh.at[ibuf.at[k]], sems, add=True).wait()

        plsc.subcore_barrier()
        pltpu.sync_copy(acc_sh.at[pl.ds(ns0, ROWS_T), :],
                        acc1.at[c, half, pl.ds(ns0, ROWS_T), :])


def _sc2_body(hw2, srcc, dstc, normr, invd,
              acc2,
              sbuf, dbuf, nb2, ivloc, rowb, zb,
              acc_sh, semg, sems):
    c = lax.axis_index("c")
    s = lax.axis_index("s")
    w = c * NS + s
    ns0 = s * ROWS_T

    def _zrow(i, _):
        zb[i, :] = _zeros16()
        return 0
    lax.fori_loop(0, K, _zrow, 0)
    for k in range(8):
        pltpu.sync_copy(zb, acc_sh.at[pl.ds(ns0 + k * K, K), :])
    plsc.subcore_barrier()

    pltpu.sync_copy(srcc.at[w], sbuf)
    pltpu.sync_copy(dstc.at[w], dbuf)
    pltpu.sync_copy(normr.at[w], nb2)
    pltpu.sync_copy(invd, ivloc)

    def _edge_batch(b, _):
        pltpu.async_copy(hw2.at[sbuf.at[b]], rowb, semg).wait()

        def _scale(jj, _):
            sc = plsc.load_gather(nb2, [_splat_i(b), _splat_i(jj)])
            rowb[jj, :] = rowb[jj, :] * sc
            return 0
        lax.fori_loop(0, K, _scale, 0)
        pltpu.async_copy(rowb, acc_sh.at[dbuf.at[b]], sems, add=True).wait()
        return 0
    lax.fori_loop(0, NB, _edge_batch, 0)

    node0 = c * (NP // 2) + s * ROWS_W
    for k in range(4):
        base = node0 + k * K
        pltpu.sync_copy(hw2.at[pl.ds(base, K), :], rowb)
        for u in range(5):
            sbuf[k, pl.ds(u * 16, 16)] = lax.iota(_i32, 16) + (base + u * 16)

        def _selfscale(jj, _):
            sp = plsc.load_gather(ivloc, [_splat_i(base + jj)])
            rowb[jj, :] = rowb[jj, :] * sp
            return 0
        lax.fori_loop(0, K, _selfscale, 0)
        pltpu.async_copy(rowb, acc_sh.at[sbuf.at[k]], sems, add=True).wait()

    plsc.subcore_barrier()
    pltpu.sync_copy(acc_sh.at[pl.ds(ns0, ROWS_T), :],
                    acc2.at[c, pl.ds(ns0, ROWS_T), :])


_SC_MESH = plsc.VectorSubcoreMesh(core_axis_name="c", subcore_axis_name="s")
_SC_PARAMS = pltpu.CompilerParams(needs_layout_passes=False,
                                  use_tc_tiling_on_sc=False)

_sc1 = pl.kernel(
    _sc1_body,
    compiler_params=_SC_PARAMS,
    out_type=[
        jax.ShapeDtypeStruct((NC, 2, NP, D // 2), _f32),
        jax.ShapeDtypeStruct((NC * NS, NB, K), _f32),
        jax.ShapeDtypeStruct((NP,), _f32),
    ],
    mesh=_SC_MESH,
    scratch_types=[
        pltpu.VMEM((NB, K), _i32),
        pltpu.VMEM((NB, K), _i32),
        pltpu.VMEM((NB, K), _f32),
        pltpu.VMEM((NB, K), _f32),
        pltpu.VMEM((NP,), _f32),
        pltpu.VMEM((K, D // 2), _f32),
        pltpu.VMEM((128, D // 2), _f32),
        pltpu.VMEM((ROWS_T,), _f32),
        pltpu.VMEM((4, K), _i32),
        pltpu.VMEM_SHARED((NP, D // 2), _f32),
        pltpu.VMEM_SHARED((NP,), _f32),
        pltpu.VMEM_SHARED((NP,), _f32),
        pltpu.SemaphoreType.DMA,
        pltpu.SemaphoreType.DMA,
    ],
)

_sc2 = pl.kernel(
    _sc2_body,
    compiler_params=_SC_PARAMS,
    out_type=jax.ShapeDtypeStruct((NC, NP, DO), _f32),
    mesh=_SC_MESH,
    scratch_types=[
        pltpu.VMEM((NB, K), _i32),
        pltpu.VMEM((NB, K), _i32),
        pltpu.VMEM((NB, K), _f32),
        pltpu.VMEM((NP,), _f32),
        pltpu.VMEM((K, DO), _f32),
        pltpu.VMEM((K, DO), _f32),
        pltpu.VMEM_SHARED((NP, DO), _f32),
        pltpu.SemaphoreType.DMA,
        pltpu.SemaphoreType.DMA,
    ],
)


_BLK = 512
_NBLK = NP // _BLK


def _mm_body(x_ref, w_ref, oa_ref, ob_ref):
    xw = jnp.dot(x_ref[...], w_ref[...], preferred_element_type=_f32)
    oa_ref[...] = xw[:, : D // 2]
    ob_ref[...] = xw[:, D // 2:]


def _tc_xw1(xp, W1):
    return pl.pallas_call(
        _mm_body,
        grid=(_NBLK,),
        in_specs=[
            pl.BlockSpec((_BLK, D), lambda i: (i, 0)),
            pl.BlockSpec((D, D), lambda i: (0, 0)),
        ],
        out_specs=[
            pl.BlockSpec((_BLK, D // 2), lambda i: (i, 0)),
            pl.BlockSpec((_BLK, D // 2), lambda i: (i, 0)),
        ],
        out_shape=[
            jax.ShapeDtypeStruct((NP, D // 2), _f32),
            jax.ShapeDtypeStruct((NP, D // 2), _f32),
        ],
    )(xp, W1)


def _relu_mm_body(acc_ref, b1_ref, w2_ref, o_ref):
    ha = jnp.maximum(acc_ref[0, 0] + acc_ref[1, 0] + b1_ref[:, : D // 2], 0.0)
    hb = jnp.maximum(acc_ref[0, 1] + acc_ref[1, 1] + b1_ref[:, D // 2:], 0.0)
    o_ref[...] = (
        jnp.dot(ha, w2_ref[: D // 2], preferred_element_type=_f32)
        + jnp.dot(hb, w2_ref[D // 2:], preferred_element_type=_f32)
    )


def _tc_hw2(acc1, b1r, W2p):
    return pl.pallas_call(
        _relu_mm_body,
        grid=(_NBLK,),
        in_specs=[
            pl.BlockSpec((NC, 2, _BLK, D // 2), lambda i: (0, 0, i, 0)),
            pl.BlockSpec((1, D), lambda i: (0, 0)),
            pl.BlockSpec((D, DO), lambda i: (0, 0)),
        ],
        out_specs=pl.BlockSpec((_BLK, DO), lambda i: (i, 0)),
        out_shape=jax.ShapeDtypeStruct((NP, DO), _f32),
    )(acc1, b1r, W2p)


def _sum_body(acc_ref, b2_ref, o_ref):
    o_ref[...] = acc_ref[0] + acc_ref[1] + b2_ref[...]


def _tc_out(acc2, b2r):
    return pl.pallas_call(
        _sum_body,
        grid=(_NBLK,),
        in_specs=[
            pl.BlockSpec((NC, _BLK, DO), lambda i: (0, i, 0)),
            pl.BlockSpec((1, DO), lambda i: (0, 0)),
        ],
        out_specs=pl.BlockSpec((_BLK, DO), lambda i: (i, 0)),
        out_shape=jax.ShapeDtypeStruct((NP, DO), _f32),
    )(acc2, b2r)


@jax.jit
def kernel(x, edge_index, edge_weight, W1, b1, W2, b2):
    src = edge_index[0]
    dst = edge_index[1]
    srcc = src.reshape(NC * NS, NB, K)
    dstc = dst.reshape(NC * NS, NB, K)
    ewc = edge_weight.reshape(NC * NS, NB, K)
    xp = jnp.pad(x, ((0, NP - N), (0, 0)))

    xw1a, xw1b = _tc_xw1(xp, W1)
    acc1, norm, invd = _sc1(xw1a, xw1b, srcc, dstc, ewc)

    hw2 = _tc_hw2(acc1, b1.reshape(1, D), jnp.pad(W2, ((0, 0), (0, DO - 5))))
    acc2 = _sc2(hw2, srcc, dstc, norm, invd)

    outp = _tc_out(acc2, jnp.pad(b2, (0, DO - 5)).reshape(1, DO))
    return outp[:N, :5]
